# baseline (speedup 1.0000x reference)
import functools
import jax
import jax.numpy as jnp
from jax import lax
from jax.experimental import pallas as pl
from jax.experimental.pallas import tpu as pltpu
from jax.experimental.pallas import tpu_sc as plsc

BLANK_ID = 0
NBEAM = 16
TBLK = 16


def _topk_body(x_ref, v_ref, c_ref):
    x = x_ref[...]
    tb, bb, cc = x.shape
    x = x.reshape(tb * bb, cc)
    iota = lax.broadcasted_iota(jnp.int32, (tb * bb, cc), 1)
    lane16 = lax.broadcasted_iota(jnp.int32, (tb * bb, NBEAM), 1)
    vacc = jnp.zeros((tb * bb, NBEAM), jnp.float32)
    cacc = jnp.zeros((tb * bb, NBEAM), jnp.int32)
    for k in range(NBEAM):
        m = jnp.max(x, axis=-1, keepdims=True)
        idx = jnp.min(jnp.where(x == m, iota, cc), axis=-1, keepdims=True)
        vacc = jnp.where(lane16 == k, m, vacc)
        cacc = jnp.where(lane16 == k, idx, cacc)
        x = jnp.where(iota == idx, -jnp.inf, x)
    v_ref[...] = vacc.reshape(tb, bb, NBEAM)
    c_ref[...] = cacc.reshape(tb, bb, NBEAM)


def _frame_topk(log_probs):
    T, B, C = log_probs.shape
    return pl.pallas_call(
        _topk_body,
        grid=(T // TBLK,),
        in_specs=[pl.BlockSpec((TBLK, B, C), lambda i: (i, 0, 0))],
        out_specs=[
            pl.BlockSpec((TBLK, B, NBEAM), lambda i: (i, 0, 0)),
            pl.BlockSpec((TBLK, B, NBEAM), lambda i: (i, 0, 0)),
        ],
        out_shape=[
            jax.ShapeDtypeStruct((T, B, NBEAM), jnp.float32),
            jax.ShapeDtypeStruct((T, B, NBEAM), jnp.int32),
        ],
    )(log_probs)


TH = 128


def _tie_utils(lane):
    even_perm = lane ^ 1
    even_first = (lane & 1) == 0
    mid = (lane >= 1) & (lane <= 14)
    odd_perm = jnp.where(mid, lane + jnp.where((lane & 1) == 1, 1, -1), lane)
    odd_first = (lane & 1) == 1
    return (even_perm, even_first), (odd_perm, odd_first)


def _make_step(tv, tc, scv, pid, cls_s, bp_s, lane, zeros16, t_off):
    (ep, ef), (op_, of) = _tie_utils(lane)

    def tie_pass(R, P, perm, first):
        scv[...] = R
        pid[...] = P
        pR = plsc.load_gather(scv, [perm])
        pP = plsc.load_gather(pid, [perm])
        swap = (R == pR) & jnp.where(first, P > pP, P < pP)
        return jnp.where(swap, pR, R), jnp.where(swap, pP, P)

    def step(t, sc):
        scv[...] = sc
        v = tv[t - t_off]
        c = tc[t - t_off]
        sc0s = plsc.load_gather(scv, [zeros16])
        R = sc0s + v
        P = c
        rvc = lax.rev(c, (0,))
        rv = lax.rev(v, (0,))
        for i in range(1, 16):
            rc = plsc.load_gather(scv, [zeros16 + i]) + rv
            rp = i * 1024 + rvc
            take_r = R >= rc
            hi = jnp.maximum(R, rc)
            hp = jnp.where(take_r, P, rp)
            R, P = plsc.sort_key_val(hi, hp, descending=True)
        R, P = tie_pass(R, P, ep, ef)
        R, P = tie_pass(R, P, op_, of)
        R, P = tie_pass(R, P, ep, ef)
        bp_s[pl.ds(t * 16, 16)] = lax.shift_right_logical(P, 10)
        cls_s[pl.ds(t * 16, 16)] = P & 1023
        return R

    return step


def _beam_a_body(tv_h, tc_h, sc_h, cls_h, bp_h,
                 tv, tc, cls_s, bp_s, scv, pid):
    b = lax.axis_index("s") * 2 + lax.axis_index("c")
    lane = lax.iota(jnp.int32, 16)
    zeros16 = jnp.zeros((16,), jnp.int32)
    pltpu.sync_copy(tv_h.at[:, b], tv)
    pltpu.sync_copy(tc_h.at[:, b], tc)
    sc0 = tv[0]
    cls_s[pl.ds(0, 16)] = tc[0]
    bp_s[pl.ds(0, 16)] = lane
    step = _make_step(tv, tc, scv, pid, cls_s, bp_s, lane, zeros16, 0)
    sc_fin = lax.fori_loop(1, TH, step, sc0)
    scv[...] = sc_fin
    pltpu.sync_copy(scv, sc_h.at[b])
    pltpu.sync_copy(cls_s, cls_h.at[b])
    pltpu.sync_copy(bp_s, bp_h.at[b])


def _beam_b_body(tv_h, tc_h, len_h, scc_h, cls_ah, bp_ah,
                 score_h, dlen_h, dec_h,
                 tv, tc, cls_s, bp_s, scv, path, dec_v, lenv, dlv, pid):
    T = dec_v.shape[0]
    b = lax.axis_index("s") * 2 + lax.axis_index("c")
    lane = lax.iota(jnp.int32, 16)
    zeros16 = jnp.zeros((16,), jnp.int32)
    pltpu.sync_copy(tv_h.at[:, b], tv)
    pltpu.sync_copy(tc_h.at[:, b], tc)
    pltpu.sync_copy(len_h, lenv)
    pltpu.sync_copy(cls_ah.at[b], cls_s.at[pl.ds(0, TH * 16)])
    pltpu.sync_copy(bp_ah.at[b], bp_s.at[pl.ds(0, TH * 16)])
    pltpu.sync_copy(scc_h.at[b], scv)
    len_b = jnp.max(plsc.load_gather(lenv, [zeros16 + b]))
    sc0 = scv[...]
    step = _make_step(tv, tc, scv, pid, cls_s, bp_s, lane, zeros16, TH)
    sc_fin = lax.fori_loop(TH, len_b, step, sc0)

    lane0 = lane == 0

    def bt(k, bm):
        t = len_b - 1 - k
        clsv = plsc.load_gather(cls_s, [t * 16 + bm])
        plsc.store_scatter(path, [zeros16 + t], clsv, mask=lane0)
        return plsc.load_gather(bp_s, [t * 16 + bm])

    lax.fori_loop(0, len_b, bt, zeros16)

    for ci in range(T // 16):
        dec_v[pl.ds(ci * 16, 16)] = zeros16

    def collapse(ci, cnt):
        tvec = ci * 16 + lane
        pc = path[pl.ds(ci * 16, 16)]
        prv = plsc.load_gather(path, [jnp.maximum(tvec - 1, 0)])
        prv = jnp.where(tvec == 0, -1, prv)
        keep = (pc != BLANK_ID) & (pc != prv) & (tvec < len_b)
        pos = plsc.cumsum(jnp.where(keep, 1, 0)) + cnt
        plsc.store_scatter(dec_v, [jnp.maximum(pos - 1, 0)], pc, mask=keep)
        return jnp.max(pos)

    cnt = lax.fori_loop(0, T // 16, collapse, jnp.int32(0))

    scv[...] = sc_fin
    dlv[...] = zeros16 + cnt
    pltpu.sync_copy(scv, score_h.at[b])
    pltpu.sync_copy(dlv, dlen_h.at[b])
    pltpu.sync_copy(dec_v, dec_h.at[b])


def _beam_first(topv, topc):
    TH_, B, _ = topv.shape
    mesh = plsc.VectorSubcoreMesh(core_axis_name="c", subcore_axis_name="s")
    run = functools.partial(
        pl.kernel,
        mesh=mesh,
        compiler_params=pltpu.CompilerParams(needs_layout_passes=False),
        out_type=[
            jax.ShapeDtypeStruct((B, 16), jnp.float32),
            jax.ShapeDtypeStruct((B, TH * 16), jnp.int32),
            jax.ShapeDtypeStruct((B, TH * 16), jnp.int32),
        ],
        scratch_types=[
            pltpu.VMEM((TH, 16), jnp.float32),
            pltpu.VMEM((TH, 16), jnp.int32),
            pltpu.VMEM((TH * 16,), jnp.int32),
            pltpu.VMEM((TH * 16,), jnp.int32),
            pltpu.VMEM((16,), jnp.float32),
            pltpu.VMEM((16,), jnp.int32),
        ],
    )(_beam_a_body)
    return run(topv, topc)


def _beam_second(topv, topc, lengths, scc, cls_a, bp_a, T):
    T2, B, _ = topv.shape
    mesh = plsc.VectorSubcoreMesh(core_axis_name="c", subcore_axis_name="s")
    run = functools.partial(
        pl.kernel,
        mesh=mesh,
        compiler_params=pltpu.CompilerParams(needs_layout_passes=False),
        out_type=[
            jax.ShapeDtypeStruct((B, 16), jnp.float32),
            jax.ShapeDtypeStruct((B, 16), jnp.int32),
            jax.ShapeDtypeStruct((B, T), jnp.int32),
        ],
        scratch_types=[
            pltpu.VMEM((T2, 16), jnp.float32),
            pltpu.VMEM((T2, 16), jnp.int32),
            pltpu.VMEM((T * 16,), jnp.int32),
            pltpu.VMEM((T * 16,), jnp.int32),
            pltpu.VMEM((16,), jnp.float32),
            pltpu.VMEM((T,), jnp.int32),
            pltpu.VMEM((T,), jnp.int32),
            pltpu.VMEM((32,), jnp.int32),
            pltpu.VMEM((16,), jnp.int32),
            pltpu.VMEM((16,), jnp.int32),
        ],
    )(_beam_b_body)
    return run(topv, topc, lengths, scc, cls_a, bp_a)


def kernel(log_probs, lengths):
    T, B, C = log_probs.shape
    tv1, tc1 = _frame_topk(log_probs[:TH])
    carry = _beam_first(tv1, tc1)
    tv2, tc2 = _frame_topk(log_probs[TH:])
    score, dlen, dec = _beam_second(tv2, tc2, lengths, *carry, T)
    return score[:, :1], dlen[:, :1], dec[:, None, :]


# speedup vs baseline: 1.1155x; 1.1155x over previous
import functools
import jax
import jax.numpy as jnp
from jax import lax
from jax.experimental import pallas as pl
from jax.experimental.pallas import tpu as pltpu
from jax.experimental.pallas import tpu_sc as plsc

BLANK_ID = 0
NBEAM = 16
TBLK = 16


def _topk_body(x_ref, v_ref, c_ref):
    x = x_ref[...]
    tb, bb, cc = x.shape
    x = x.reshape(tb * bb, cc)
    iota = lax.broadcasted_iota(jnp.int32, (tb * bb, cc), 1)
    lane16 = lax.broadcasted_iota(jnp.int32, (tb * bb, NBEAM), 1)
    vacc = jnp.zeros((tb * bb, NBEAM), jnp.float32)
    cacc = jnp.zeros((tb * bb, NBEAM), jnp.int32)
    for k in range(NBEAM):
        m = jnp.max(x, axis=-1, keepdims=True)
        idx = jnp.min(jnp.where(x == m, iota, cc), axis=-1, keepdims=True)
        vacc = jnp.where(lane16 == k, m, vacc)
        cacc = jnp.where(lane16 == k, idx, cacc)
        x = jnp.where(iota == idx, -jnp.inf, x)
    v_ref[...] = vacc.reshape(tb, bb, NBEAM)
    c_ref[...] = cacc.reshape(tb, bb, NBEAM)


def _frame_topk(log_probs, t0, nt):
    T, B, C = log_probs.shape
    off = t0 // TBLK
    return pl.pallas_call(
        _topk_body,
        grid=(nt // TBLK,),
        in_specs=[pl.BlockSpec((TBLK, B, C), lambda i: (i + off, 0, 0))],
        out_specs=[
            pl.BlockSpec((TBLK, B, NBEAM), lambda i: (i, 0, 0)),
            pl.BlockSpec((TBLK, B, NBEAM), lambda i: (i, 0, 0)),
        ],
        out_shape=[
            jax.ShapeDtypeStruct((nt, B, NBEAM), jnp.float32),
            jax.ShapeDtypeStruct((nt, B, NBEAM), jnp.int32),
        ],
    )(log_probs)


TH = 128


def _tie_utils(lane):
    even_perm = lane ^ 1
    even_first = (lane & 1) == 0
    mid = (lane >= 1) & (lane <= 14)
    odd_perm = jnp.where(mid, lane + jnp.where((lane & 1) == 1, 1, -1), lane)
    odd_first = (lane & 1) == 1
    return (even_perm, even_first), (odd_perm, odd_first)


def _make_step(tv, tc, scv, pid, cls_s, bp_s, lane, zeros16, t_off):
    (ep, ef), (op_, of) = _tie_utils(lane)

    def tie_pass(R, P, perm, first):
        scv[...] = R
        pid[...] = P
        pR = plsc.load_gather(scv, [perm])
        pP = plsc.load_gather(pid, [perm])
        swap = (R == pR) & jnp.where(first, P > pP, P < pP)
        return jnp.where(swap, pR, R), jnp.where(swap, pP, P)

    def step(t, sc):
        scv[...] = sc
        v = tv[t - t_off]
        c = tc[t - t_off]
        sc0s = plsc.load_gather(scv, [zeros16])
        R = sc0s + v
        P = c
        rvc = lax.rev(c, (0,))
        rv = lax.rev(v, (0,))
        for i in range(1, 16):
            rc = plsc.load_gather(scv, [zeros16 + i]) + rv
            rp = i * 1024 + rvc
            take_r = R >= rc
            hi = jnp.maximum(R, rc)
            hp = jnp.where(take_r, P, rp)
            R, P = plsc.sort_key_val(hi, hp, descending=True)
        R, P = tie_pass(R, P, ep, ef)
        R, P = tie_pass(R, P, op_, of)
        R, P = tie_pass(R, P, ep, ef)
        bp_s[pl.ds(t * 16, 16)] = lax.shift_right_logical(P, 10)
        cls_s[pl.ds(t * 16, 16)] = P & 1023
        return R

    return step


def _beam_a_body(tv_h, tc_h, sc_h, cls_h, bp_h,
                 tv, tc, cls_s, bp_s, scv, pid):
    b = lax.axis_index("s") * 2 + lax.axis_index("c")
    lane = lax.iota(jnp.int32, 16)
    zeros16 = jnp.zeros((16,), jnp.int32)
    pltpu.sync_copy(tv_h.at[:, b], tv)
    pltpu.sync_copy(tc_h.at[:, b], tc)
    sc0 = tv[0]
    cls_s[pl.ds(0, 16)] = tc[0]
    bp_s[pl.ds(0, 16)] = lane
    step = _make_step(tv, tc, scv, pid, cls_s, bp_s, lane, zeros16, 0)
    sc_fin = lax.fori_loop(1, TH, step, sc0)
    scv[...] = sc_fin
    pltpu.sync_copy(scv, sc_h.at[b])
    pltpu.sync_copy(cls_s, cls_h.at[b])
    pltpu.sync_copy(bp_s, bp_h.at[b])


def _beam_b_body(tv_h, tc_h, len_h, scc_h, cls_ah, bp_ah,
                 score_h, dlen_h, dec_h,
                 tv, tc, cls_s, bp_s, scv, path, dec_v, lenv, dlv, pid):
    T = dec_v.shape[0]
    b = lax.axis_index("s") * 2 + lax.axis_index("c")
    lane = lax.iota(jnp.int32, 16)
    zeros16 = jnp.zeros((16,), jnp.int32)
    pltpu.sync_copy(tv_h.at[:, b], tv)
    pltpu.sync_copy(tc_h.at[:, b], tc)
    pltpu.sync_copy(len_h, lenv)
    pltpu.sync_copy(cls_ah.at[b], cls_s.at[pl.ds(0, TH * 16)])
    pltpu.sync_copy(bp_ah.at[b], bp_s.at[pl.ds(0, TH * 16)])
    pltpu.sync_copy(scc_h.at[b], scv)
    len_b = jnp.max(plsc.load_gather(lenv, [zeros16 + b]))
    sc0 = scv[...]
    step = _make_step(tv, tc, scv, pid, cls_s, bp_s, lane, zeros16, TH)
    sc_fin = lax.fori_loop(TH, len_b, step, sc0)

    lane0 = lane == 0

    def bt(k, bm):
        t = len_b - 1 - k
        clsv = plsc.load_gather(cls_s, [t * 16 + bm])
        plsc.store_scatter(path, [zeros16 + t], clsv, mask=lane0)
        return plsc.load_gather(bp_s, [t * 16 + bm])

    lax.fori_loop(0, len_b, bt, zeros16)

    for ci in range(T // 16):
        dec_v[pl.ds(ci * 16, 16)] = zeros16

    def collapse(ci, cnt):
        tvec = ci * 16 + lane
        pc = path[pl.ds(ci * 16, 16)]
        prv = plsc.load_gather(path, [jnp.maximum(tvec - 1, 0)])
        prv = jnp.where(tvec == 0, -1, prv)
        keep = (pc != BLANK_ID) & (pc != prv) & (tvec < len_b)
        pos = plsc.cumsum(jnp.where(keep, 1, 0)) + cnt
        plsc.store_scatter(dec_v, [jnp.maximum(pos - 1, 0)], pc, mask=keep)
        return jnp.max(pos)

    cnt = lax.fori_loop(0, T // 16, collapse, jnp.int32(0))

    scv[...] = sc_fin
    dlv[...] = zeros16 + cnt
    pltpu.sync_copy(scv, score_h.at[b])
    pltpu.sync_copy(dlv, dlen_h.at[b])
    pltpu.sync_copy(dec_v, dec_h.at[b])


def _beam_first(topv, topc):
    TH_, B, _ = topv.shape
    mesh = plsc.VectorSubcoreMesh(core_axis_name="c", subcore_axis_name="s")
    run = functools.partial(
        pl.kernel,
        mesh=mesh,
        compiler_params=pltpu.CompilerParams(needs_layout_passes=False),
        out_type=[
            jax.ShapeDtypeStruct((B, 16), jnp.float32),
            jax.ShapeDtypeStruct((B, TH * 16), jnp.int32),
            jax.ShapeDtypeStruct((B, TH * 16), jnp.int32),
        ],
        scratch_types=[
            pltpu.VMEM((TH, 16), jnp.float32),
            pltpu.VMEM((TH, 16), jnp.int32),
            pltpu.VMEM((TH * 16,), jnp.int32),
            pltpu.VMEM((TH * 16,), jnp.int32),
            pltpu.VMEM((16,), jnp.float32),
            pltpu.VMEM((16,), jnp.int32),
        ],
    )(_beam_a_body)
    return run(topv, topc)


def _beam_second(topv, topc, lengths, scc, cls_a, bp_a, T):
    T2, B, _ = topv.shape
    mesh = plsc.VectorSubcoreMesh(core_axis_name="c", subcore_axis_name="s")
    run = functools.partial(
        pl.kernel,
        mesh=mesh,
        compiler_params=pltpu.CompilerParams(needs_layout_passes=False),
        out_type=[
            jax.ShapeDtypeStruct((B, 16), jnp.float32),
            jax.ShapeDtypeStruct((B, 16), jnp.int32),
            jax.ShapeDtypeStruct((B, T), jnp.int32),
        ],
        scratch_types=[
            pltpu.VMEM((T2, 16), jnp.float32),
            pltpu.VMEM((T2, 16), jnp.int32),
            pltpu.VMEM((T * 16,), jnp.int32),
            pltpu.VMEM((T * 16,), jnp.int32),
            pltpu.VMEM((16,), jnp.float32),
            pltpu.VMEM((T,), jnp.int32),
            pltpu.VMEM((T,), jnp.int32),
            pltpu.VMEM((32,), jnp.int32),
            pltpu.VMEM((16,), jnp.int32),
            pltpu.VMEM((16,), jnp.int32),
        ],
    )(_beam_b_body)
    return run(topv, topc, lengths, scc, cls_a, bp_a)


def kernel(log_probs, lengths):
    T, B, C = log_probs.shape
    tv1, tc1 = _frame_topk(log_probs, 0, TH)
    carry = _beam_first(tv1, tc1)
    tv2, tc2 = _frame_topk(log_probs, TH, T - TH)
    score, dlen, dec = _beam_second(tv2, tc2, lengths, *carry, T)
    return score[:, :1], dlen[:, :1], dec[:, None, :]


# speedup vs baseline: 1.1626x; 1.0422x over previous
import functools
import jax
import jax.numpy as jnp
from jax import lax
from jax.experimental import pallas as pl
from jax.experimental.pallas import tpu as pltpu
from jax.experimental.pallas import tpu_sc as plsc

BLANK_ID = 0
NBEAM = 16
TBLK = 16


def _topk_body(x_ref, v_ref, c_ref):
    x = x_ref[...]
    tb, bb, cc = x.shape
    x = x.reshape(tb * bb, cc)
    iota = lax.broadcasted_iota(jnp.int32, (tb * bb, cc), 1)
    lane16 = lax.broadcasted_iota(jnp.int32, (tb * bb, NBEAM), 1)
    vacc = jnp.zeros((tb * bb, NBEAM), jnp.float32)
    cacc = jnp.zeros((tb * bb, NBEAM), jnp.int32)
    for k in range(NBEAM):
        m = jnp.max(x, axis=-1, keepdims=True)
        idx = jnp.min(jnp.where(x == m, iota, cc), axis=-1, keepdims=True)
        vacc = jnp.where(lane16 == k, m, vacc)
        cacc = jnp.where(lane16 == k, idx, cacc)
        x = jnp.where(iota == idx, -jnp.inf, x)
    v_ref[...] = vacc.reshape(tb, bb, NBEAM)
    c_ref[...] = cacc.reshape(tb, bb, NBEAM)


def _frame_topk(log_probs, t0, nt):
    T, B, C = log_probs.shape
    off = t0 // TBLK
    return pl.pallas_call(
        _topk_body,
        grid=(nt // TBLK,),
        in_specs=[pl.BlockSpec((TBLK, B, C), lambda i: (i + off, 0, 0))],
        out_specs=[
            pl.BlockSpec((TBLK, B, NBEAM), lambda i: (i, 0, 0)),
            pl.BlockSpec((TBLK, B, NBEAM), lambda i: (i, 0, 0)),
        ],
        out_shape=[
            jax.ShapeDtypeStruct((nt, B, NBEAM), jnp.float32),
            jax.ShapeDtypeStruct((nt, B, NBEAM), jnp.int32),
        ],
    )(log_probs)


TH = 128
TM = 192


def _tie_utils(lane):
    even_perm = lane ^ 1
    even_first = (lane & 1) == 0
    mid = (lane >= 1) & (lane <= 14)
    odd_perm = jnp.where(mid, lane + jnp.where((lane & 1) == 1, 1, -1), lane)
    odd_first = (lane & 1) == 1
    return (even_perm, even_first), (odd_perm, odd_first)


def _make_step(tv, tc, scv, pid, cls_s, bp_s, lane, zeros16, t_off, s_off):
    (ep, ef), (op_, of) = _tie_utils(lane)

    def tie_pass(R, P, perm, first):
        scv[...] = R
        pid[...] = P
        pR = plsc.load_gather(scv, [perm])
        pP = plsc.load_gather(pid, [perm])
        swap = (R == pR) & jnp.where(first, P > pP, P < pP)
        return jnp.where(swap, pR, R), jnp.where(swap, pP, P)

    def step(t, sc):
        scv[...] = sc
        v = tv[t - t_off]
        c = tc[t - t_off]
        sc0s = plsc.load_gather(scv, [zeros16])
        R = sc0s + v
        P = c
        rvc = lax.rev(c, (0,))
        rv = lax.rev(v, (0,))
        for i in range(1, 16):
            rc = plsc.load_gather(scv, [zeros16 + i]) + rv
            rp = i * 1024 + rvc
            take_r = R >= rc
            hi = jnp.maximum(R, rc)
            hp = jnp.where(take_r, P, rp)
            R, P = plsc.sort_key_val(hi, hp, descending=True)
        R, P = tie_pass(R, P, ep, ef)
        R, P = tie_pass(R, P, op_, of)
        R, P = tie_pass(R, P, ep, ef)
        bp_s[pl.ds((t - s_off) * 16, 16)] = lax.shift_right_logical(P, 10)
        cls_s[pl.ds((t - s_off) * 16, 16)] = P & 1023
        return R

    return step


def _beam_a_body(tv_h, tc_h, sc_h, cls_h, bp_h,
                 tv, tc, cls_s, bp_s, scv, pid):
    b = lax.axis_index("s") * 2 + lax.axis_index("c")
    lane = lax.iota(jnp.int32, 16)
    zeros16 = jnp.zeros((16,), jnp.int32)
    pltpu.sync_copy(tv_h.at[:, b], tv)
    pltpu.sync_copy(tc_h.at[:, b], tc)
    sc0 = tv[0]
    cls_s[pl.ds(0, 16)] = tc[0]
    bp_s[pl.ds(0, 16)] = lane
    step = _make_step(tv, tc, scv, pid, cls_s, bp_s, lane, zeros16, 0, 0)
    sc_fin = lax.fori_loop(1, TH, step, sc0)
    scv[...] = sc_fin
    pltpu.sync_copy(scv, sc_h.at[b])
    pltpu.sync_copy(cls_s, cls_h.at[b])
    pltpu.sync_copy(bp_s, bp_h.at[b])


def _beam_m_body(tv_h, tc_h, len_h, scin_h, sc_h, cls_h, bp_h,
                 tv, tc, cls_s, bp_s, scv, lenv, pid):
    b = lax.axis_index("s") * 2 + lax.axis_index("c")
    lane = lax.iota(jnp.int32, 16)
    zeros16 = jnp.zeros((16,), jnp.int32)
    pltpu.sync_copy(tv_h.at[:, b], tv)
    pltpu.sync_copy(tc_h.at[:, b], tc)
    pltpu.sync_copy(len_h, lenv)
    pltpu.sync_copy(scin_h.at[b], scv)
    len_b = jnp.max(plsc.load_gather(lenv, [zeros16 + b]))
    sc0 = scv[...]
    step = _make_step(tv, tc, scv, pid, cls_s, bp_s, lane, zeros16, TH, TH)
    sc_fin = lax.fori_loop(TH, jnp.minimum(len_b, TM), step, sc0)
    scv[...] = sc_fin
    pltpu.sync_copy(scv, sc_h.at[b])
    pltpu.sync_copy(cls_s, cls_h.at[b])
    pltpu.sync_copy(bp_s, bp_h.at[b])


def _beam_f_body(tv_h, tc_h, len_h, scin_h, cls_ah, bp_ah, cls_mh, bp_mh,
                 score_h, dlen_h, dec_h,
                 tv, tc, cls_s, bp_s, scv, path, dec_v, lenv, dlv, pid):
    T = dec_v.shape[0]
    b = lax.axis_index("s") * 2 + lax.axis_index("c")
    lane = lax.iota(jnp.int32, 16)
    zeros16 = jnp.zeros((16,), jnp.int32)
    pltpu.sync_copy(tv_h.at[:, b], tv)
    pltpu.sync_copy(tc_h.at[:, b], tc)
    pltpu.sync_copy(len_h, lenv)
    pltpu.sync_copy(cls_ah.at[b], cls_s.at[pl.ds(0, TH * 16)])
    pltpu.sync_copy(bp_ah.at[b], bp_s.at[pl.ds(0, TH * 16)])
    pltpu.sync_copy(cls_mh.at[b], cls_s.at[pl.ds(TH * 16, (TM - TH) * 16)])
    pltpu.sync_copy(bp_mh.at[b], bp_s.at[pl.ds(TH * 16, (TM - TH) * 16)])
    pltpu.sync_copy(scin_h.at[b], scv)
    len_b = jnp.max(plsc.load_gather(lenv, [zeros16 + b]))
    sc0 = scv[...]
    step = _make_step(tv, tc, scv, pid, cls_s, bp_s, lane, zeros16, TM, 0)
    sc_fin = lax.fori_loop(TM, jnp.maximum(len_b, TM), step, sc0)

    lane0 = lane == 0

    def bt(k, bm):
        t = len_b - 1 - k
        clsv = plsc.load_gather(cls_s, [t * 16 + bm])
        plsc.store_scatter(path, [zeros16 + t], clsv, mask=lane0)
        return plsc.load_gather(bp_s, [t * 16 + bm])

    lax.fori_loop(0, len_b, bt, zeros16)

    for ci in range(T // 16):
        dec_v[pl.ds(ci * 16, 16)] = zeros16

    def collapse(ci, cnt):
        tvec = ci * 16 + lane
        pc = path[pl.ds(ci * 16, 16)]
        prv = plsc.load_gather(path, [jnp.maximum(tvec - 1, 0)])
        prv = jnp.where(tvec == 0, -1, prv)
        keep = (pc != BLANK_ID) & (pc != prv) & (tvec < len_b)
        pos = plsc.cumsum(jnp.where(keep, 1, 0)) + cnt
        plsc.store_scatter(dec_v, [jnp.maximum(pos - 1, 0)], pc, mask=keep)
        return jnp.max(pos)

    cnt = lax.fori_loop(0, T // 16, collapse, jnp.int32(0))

    scv[...] = sc_fin
    dlv[...] = zeros16 + cnt
    pltpu.sync_copy(scv, score_h.at[b])
    pltpu.sync_copy(dlv, dlen_h.at[b])
    pltpu.sync_copy(dec_v, dec_h.at[b])


def _sc_mesh_kernel(body, out_type, scratch_types):
    mesh = plsc.VectorSubcoreMesh(core_axis_name="c", subcore_axis_name="s")
    return pl.kernel(
        body,
        mesh=mesh,
        compiler_params=pltpu.CompilerParams(needs_layout_passes=False),
        out_type=out_type,
        scratch_types=scratch_types,
    )


def _beam_first(topv, topc):
    B = topv.shape[1]
    run = _sc_mesh_kernel(
        _beam_a_body,
        [
            jax.ShapeDtypeStruct((B, 16), jnp.float32),
            jax.ShapeDtypeStruct((B, TH * 16), jnp.int32),
            jax.ShapeDtypeStruct((B, TH * 16), jnp.int32),
        ],
        [
            pltpu.VMEM((TH, 16), jnp.float32),
            pltpu.VMEM((TH, 16), jnp.int32),
            pltpu.VMEM((TH * 16,), jnp.int32),
            pltpu.VMEM((TH * 16,), jnp.int32),
            pltpu.VMEM((16,), jnp.float32),
            pltpu.VMEM((16,), jnp.int32),
        ],
    )
    return run(topv, topc)


def _beam_mid(topv, topc, lengths, scc):
    NS = TM - TH
    B = topv.shape[1]
    run = _sc_mesh_kernel(
        _beam_m_body,
        [
            jax.ShapeDtypeStruct((B, 16), jnp.float32),
            jax.ShapeDtypeStruct((B, NS * 16), jnp.int32),
            jax.ShapeDtypeStruct((B, NS * 16), jnp.int32),
        ],
        [
            pltpu.VMEM((NS, 16), jnp.float32),
            pltpu.VMEM((NS, 16), jnp.int32),
            pltpu.VMEM((NS * 16,), jnp.int32),
            pltpu.VMEM((NS * 16,), jnp.int32),
            pltpu.VMEM((16,), jnp.float32),
            pltpu.VMEM((32,), jnp.int32),
            pltpu.VMEM((16,), jnp.int32),
        ],
    )
    return run(topv, topc, lengths, scc)


def _beam_final(topv, topc, lengths, scc, cls_a, bp_a, cls_m, bp_m, T):
    NS = T - TM
    B = topv.shape[1]
    run = _sc_mesh_kernel(
        _beam_f_body,
        [
            jax.ShapeDtypeStruct((B, 16), jnp.float32),
            jax.ShapeDtypeStruct((B, 16), jnp.int32),
            jax.ShapeDtypeStruct((B, T), jnp.int32),
        ],
        [
            pltpu.VMEM((NS, 16), jnp.float32),
            pltpu.VMEM((NS, 16), jnp.int32),
            pltpu.VMEM((T * 16,), jnp.int32),
            pltpu.VMEM((T * 16,), jnp.int32),
            pltpu.VMEM((16,), jnp.float32),
            pltpu.VMEM((T,), jnp.int32),
            pltpu.VMEM((T,), jnp.int32),
            pltpu.VMEM((32,), jnp.int32),
            pltpu.VMEM((16,), jnp.int32),
            pltpu.VMEM((16,), jnp.int32),
        ],
    )
    return run(topv, topc, lengths, scc, cls_a, bp_a, cls_m, bp_m)


def kernel(log_probs, lengths):
    T, B, C = log_probs.shape
    tv1, tc1 = _frame_topk(log_probs, 0, TH)
    sc1, cls1, bp1 = _beam_first(tv1, tc1)
    tv2, tc2 = _frame_topk(log_probs, TH, TM - TH)
    sc2, cls2, bp2 = _beam_mid(tv2, tc2, lengths, sc1)
    tv3, tc3 = _frame_topk(log_probs, TM, T - TM)
    score, dlen, dec = _beam_final(
        tv3, tc3, lengths, sc2, cls1, bp1, cls2, bp2, T)
    return score[:, :1], dlen[:, :1], dec[:, None, :]
